# baseline (speedup 1.0000x reference)
import jax
import jax.numpy as jnp
from jax.experimental import pallas as pl
from jax.experimental.pallas import tpu as pltpu

N = 10000
BM = 200


def _s0_kernel(x_ref, w0_ref, o_ref):
    o_ref[...] = jnp.dot(
        x_ref[...].astype(jnp.bfloat16), w0_ref[...].astype(jnp.bfloat16),
        preferred_element_type=jnp.float32).astype(jnp.bfloat16)


def _pass1_kernel(adj_ref, s0_ref, b0_ref, w1_ref, o_ref):
    g = jnp.dot(adj_ref[...].astype(jnp.bfloat16), s0_ref[...],
                preferred_element_type=jnp.float32)
    h = jnp.maximum(g + b0_ref[...], 0.0).astype(jnp.bfloat16)
    o_ref[...] = jnp.dot(h, w1_ref[...].astype(jnp.bfloat16),
                         preferred_element_type=jnp.float32).astype(jnp.bfloat16)


def _pass2_kernel(adj_ref, s1_ref, b1_ref, o_ref):
    z = jnp.dot(adj_ref[...].astype(jnp.bfloat16), s1_ref[...],
                preferred_element_type=jnp.float32)
    z = z + b1_ref[...]
    m = jnp.max(z, axis=-1, keepdims=True)
    z = z - m
    lse = jnp.log(jnp.sum(jnp.exp(z), axis=-1, keepdims=True))
    o_ref[...] = z - lse


@jax.jit
def kernel(x, adj, W0, b0, W1, b1):
    nfeat = x.shape[1]
    nhid = W0.shape[1]
    ncls = W1.shape[1]

    s0 = pl.pallas_call(
        _s0_kernel,
        grid=(N // 2000,),
        in_specs=[
            pl.BlockSpec((2000, nfeat), lambda i: (i, 0)),
            pl.BlockSpec((nfeat, nhid), lambda i: (0, 0)),
        ],
        out_specs=pl.BlockSpec((2000, nhid), lambda i: (i, 0)),
        out_shape=jax.ShapeDtypeStruct((N, nhid), jnp.bfloat16),
    )(x, W0)

    s1 = pl.pallas_call(
        _pass1_kernel,
        grid=(N // BM,),
        in_specs=[
            pl.BlockSpec((BM, N), lambda i: (i, 0)),
            pl.BlockSpec((N, nhid), lambda i: (0, 0)),
            pl.BlockSpec((1, nhid), lambda i: (0, 0)),
            pl.BlockSpec((nhid, ncls), lambda i: (0, 0)),
        ],
        out_specs=pl.BlockSpec((BM, ncls), lambda i: (i, 0)),
        out_shape=jax.ShapeDtypeStruct((N, ncls), jnp.bfloat16),
        compiler_params=pltpu.CompilerParams(
            dimension_semantics=("arbitrary",)),
    )(adj, s0, b0.reshape(1, nhid), W1)

    out = pl.pallas_call(
        _pass2_kernel,
        grid=(N // BM,),
        in_specs=[
            pl.BlockSpec((BM, N), lambda i: (i, 0)),
            pl.BlockSpec((N, ncls), lambda i: (0, 0)),
            pl.BlockSpec((1, ncls), lambda i: (0, 0)),
        ],
        out_specs=pl.BlockSpec((BM, ncls), lambda i: (i, 0)),
        out_shape=jax.ShapeDtypeStruct((N, ncls), jnp.float32),
        compiler_params=pltpu.CompilerParams(
            dimension_semantics=("arbitrary",)),
    )(adj, s1, b1.reshape(1, ncls))

    return out


# speedup vs baseline: 1.0291x; 1.0291x over previous
import jax
import jax.numpy as jnp
from jax.experimental import pallas as pl
from jax.experimental.pallas import tpu as pltpu

N = 10000
BM = 400


def _s0_kernel(x_ref, w0_ref, o_ref):
    o_ref[...] = jnp.dot(
        x_ref[...].astype(jnp.bfloat16), w0_ref[...].astype(jnp.bfloat16),
        preferred_element_type=jnp.float32).astype(jnp.bfloat16)


def _pass1_kernel(adj_ref, s0_ref, b0_ref, w1_ref, o_ref):
    g = jnp.dot(adj_ref[...].astype(jnp.bfloat16), s0_ref[...],
                preferred_element_type=jnp.float32)
    h = jnp.maximum(g + b0_ref[...], 0.0).astype(jnp.bfloat16)
    o_ref[...] = jnp.dot(h, w1_ref[...].astype(jnp.bfloat16),
                         preferred_element_type=jnp.float32).astype(jnp.bfloat16)


def _pass2_kernel(adj_ref, s1_ref, b1_ref, o_ref):
    z = jnp.dot(adj_ref[...].astype(jnp.bfloat16), s1_ref[...],
                preferred_element_type=jnp.float32)
    z = z + b1_ref[...]
    m = jnp.max(z, axis=-1, keepdims=True)
    z = z - m
    lse = jnp.log(jnp.sum(jnp.exp(z), axis=-1, keepdims=True))
    o_ref[...] = z - lse


@jax.jit
def kernel(x, adj, W0, b0, W1, b1):
    nfeat = x.shape[1]
    nhid = W0.shape[1]
    ncls = W1.shape[1]

    s0 = pl.pallas_call(
        _s0_kernel,
        grid=(N // 2000,),
        in_specs=[
            pl.BlockSpec((2000, nfeat), lambda i: (i, 0)),
            pl.BlockSpec((nfeat, nhid), lambda i: (0, 0)),
        ],
        out_specs=pl.BlockSpec((2000, nhid), lambda i: (i, 0)),
        out_shape=jax.ShapeDtypeStruct((N, nhid), jnp.bfloat16),
    )(x, W0)

    s1 = pl.pallas_call(
        _pass1_kernel,
        grid=(N // BM,),
        in_specs=[
            pl.BlockSpec((BM, N), lambda i: (i, 0)),
            pl.BlockSpec((N, nhid), lambda i: (0, 0)),
            pl.BlockSpec((1, nhid), lambda i: (0, 0)),
            pl.BlockSpec((nhid, ncls), lambda i: (0, 0)),
        ],
        out_specs=pl.BlockSpec((BM, ncls), lambda i: (i, 0)),
        out_shape=jax.ShapeDtypeStruct((N, ncls), jnp.bfloat16),
        compiler_params=pltpu.CompilerParams(
            dimension_semantics=("arbitrary",)),
    )(adj, s0, b0.reshape(1, nhid), W1)

    out = pl.pallas_call(
        _pass2_kernel,
        grid=(N // BM,),
        in_specs=[
            pl.BlockSpec((BM, N), lambda i: (i, 0)),
            pl.BlockSpec((N, ncls), lambda i: (0, 0)),
            pl.BlockSpec((1, ncls), lambda i: (0, 0)),
        ],
        out_specs=pl.BlockSpec((BM, ncls), lambda i: (i, 0)),
        out_shape=jax.ShapeDtypeStruct((N, ncls), jnp.float32),
        compiler_params=pltpu.CompilerParams(
            dimension_semantics=("arbitrary",)),
    )(adj, s1, b1.reshape(1, ncls))

    return out
